# speedup vs baseline: 1.1746x; 1.1746x over previous
import jax
import jax.numpy as jnp
from jax.experimental import pallas as pl
from jax.experimental.pallas import tpu as pltpu

B, S, D, H = 2, 2048, 2048, 16
DH = D // H
HALF = DH // 2
BS = B * S
ROPE_THETA = 10000.0
LN_EPS = 1e-5


BM1 = 1024
BN1 = 512
NSEC = D // BN1


def _ln_qkv_rope_kernel(x_ref, w_ref, cs_ref, o_ref, xn_ref):
    j = pl.program_id(1)

    @pl.when(j == 0)
    def _():
        xb = x_ref[...]
        mu = jnp.mean(xb, axis=-1, keepdims=True)
        xc = xb - mu
        var = jnp.mean(xc * xc, axis=-1, keepdims=True)
        xn_ref[...] = xc * jax.lax.rsqrt(var + LN_EPS)

    y = jnp.dot(xn_ref[...], w_ref[...], preferred_element_type=jnp.float32)

    @pl.when(j < 2 * NSEC)
    def _():
        cos = cs_ref[:, :HALF]
        sin = cs_ref[:, HALF:]
        parts = []
        for h in range(BN1 // DH):
            yh = y[:, h * DH:(h + 1) * DH]
            x1 = yh[:, :HALF]
            x2 = yh[:, HALF:]
            parts.append(jnp.concatenate(
                [x1 * cos - x2 * sin, x2 * cos + x1 * sin], axis=-1))
        o_ref[0] = jnp.concatenate(parts, axis=-1)

    @pl.when(j >= 2 * NSEC)
    def _():
        o_ref[0] = y


def _ln_qkv_rope(x2, w_qkv, cs):
    return pl.pallas_call(
        _ln_qkv_rope_kernel,
        out_shape=jax.ShapeDtypeStruct((3, BS, D), jnp.float32),
        grid=(BS // BM1, (3 * D) // BN1),
        in_specs=[
            pl.BlockSpec((BM1, D), lambda i, j: (i, 0)),
            pl.BlockSpec((D, BN1), lambda i, j: (0, j)),
            pl.BlockSpec((BM1, DH), lambda i, j: (i, 0)),
        ],
        out_specs=pl.BlockSpec((1, BM1, BN1),
                               lambda i, j: (j // NSEC, i, j % NSEC)),
        scratch_shapes=[pltpu.VMEM((BM1, D), jnp.float32)],
        compiler_params=pltpu.CompilerParams(
            dimension_semantics=("parallel", "arbitrary"),
            vmem_limit_bytes=50 * 1024 * 1024,
        ),
        name="ln_qkv_rope",
    )(x2, w_qkv, cs)


BQ = 512
BK = 512
NKC = S // BK


def _attn_kernel(q_ref, k_ref, v_ref, o_ref, acc_ref, m_ref, l_ref):
    qi = pl.program_id(1)
    q = q_ref[0]
    acc_ref[...] = jnp.zeros_like(acc_ref)
    m_ref[...] = jnp.full_like(m_ref, -1e30)
    l_ref[...] = jnp.zeros_like(l_ref)
    rows = qi * BQ + jax.lax.broadcasted_iota(jnp.int32, (BQ, 1), 0)
    scale = DH ** -0.5
    for kc in range(NKC):
        @pl.when(kc <= qi)
        def _(kc=kc):
            k = k_ref[0, kc * BK:(kc + 1) * BK, :]
            v = v_ref[0, kc * BK:(kc + 1) * BK, :]
            s = jax.lax.dot_general(
                q, k, (((1,), (1,)), ((), ())),
                preferred_element_type=jnp.float32) * scale
            cols = kc * BK + jax.lax.broadcasted_iota(jnp.int32, (1, BK), 1)
            s = jnp.where(rows >= cols, s, -1e30)
            m_prev = m_ref[:, 0:1]
            l_prev = l_ref[:, 0:1]
            m_new = jnp.maximum(m_prev, jnp.max(s, axis=-1, keepdims=True))
            alpha = jnp.exp(m_prev - m_new)
            p = jnp.exp(s - m_new)
            l_new = l_prev * alpha + jnp.sum(p, axis=-1, keepdims=True)
            acc_ref[...] = acc_ref[...] * alpha + jnp.dot(
                p, v, preferred_element_type=jnp.float32)
            m_ref[...] = jnp.broadcast_to(m_new, m_ref.shape)
            l_ref[...] = jnp.broadcast_to(l_new, l_ref.shape)
    o_ref[0] = acc_ref[...] / l_ref[:, 0:1]


def _flash_attn(q3, k3, v3):
    return pl.pallas_call(
        _attn_kernel,
        out_shape=jax.ShapeDtypeStruct((B, S, D), jnp.float32),
        grid=(B * H, S // BQ),
        in_specs=[
            pl.BlockSpec((1, BQ, DH), lambda bh, qi: (bh // H, qi, bh % H)),
            pl.BlockSpec((1, S, DH), lambda bh, qi: (bh // H, 0, bh % H)),
            pl.BlockSpec((1, S, DH), lambda bh, qi: (bh // H, 0, bh % H)),
        ],
        out_specs=pl.BlockSpec((1, BQ, DH),
                               lambda bh, qi: (bh // H, qi, bh % H)),
        scratch_shapes=[
            pltpu.VMEM((BQ, DH), jnp.float32),
            pltpu.VMEM((BQ, DH), jnp.float32),
            pltpu.VMEM((BQ, DH), jnp.float32),
        ],
        compiler_params=pltpu.CompilerParams(
            dimension_semantics=("parallel", "arbitrary"),
        ),
        name="flash_attn",
    )(q3, k3, v3)


BM3 = 256


def _proj_kernel(x_ref, w_ref, o_ref):
    o_ref[...] = jnp.dot(x_ref[...], w_ref[...],
                         preferred_element_type=jnp.float32)


def _out_proj(x2, w_out):
    return pl.pallas_call(
        _proj_kernel,
        out_shape=jax.ShapeDtypeStruct((BS, D), jnp.float32),
        grid=(BS // BM3,),
        in_specs=[
            pl.BlockSpec((BM3, D), lambda i: (i, 0)),
            pl.BlockSpec((D, D), lambda i: (0, 0)),
        ],
        out_specs=pl.BlockSpec((BM3, D), lambda i: (i, 0)),
        compiler_params=pltpu.CompilerParams(
            dimension_semantics=("parallel",),
            vmem_limit_bytes=50 * 1024 * 1024,
        ),
        name="out_proj",
    )(x2, w_out)


def kernel(positions, hidden_states, w_qkv, w_out):
    x2 = hidden_states.reshape(BS, D)
    pos_f = positions.reshape(BS).astype(jnp.float32)
    inv_freq = 1.0 / (ROPE_THETA ** (
        jnp.arange(HALF, dtype=jnp.float32) / HALF))
    ang = pos_f[:, None] * inv_freq[None, :]
    cs = jnp.concatenate([jnp.cos(ang), jnp.sin(ang)], axis=-1)

    qkv3 = _ln_qkv_rope(x2, w_qkv, cs)
    q3 = qkv3[0].reshape(B, S, D)
    k3 = qkv3[1].reshape(B, S, D)
    v3 = qkv3[2].reshape(B, S, D)

    attn = _flash_attn(q3, k3, v3)
    out = _out_proj(attn.reshape(BS, D), w_out)
    return out.reshape(B, S, D)
